# baseline (speedup 1.0000x reference)
import functools

import jax
import jax.numpy as jnp
from jax import lax
from jax.experimental import pallas as pl
from jax.experimental.pallas import tpu as pltpu
from jax.experimental.pallas import tpu_sc as plsc

N = 8192
B = 16384
NC, NS = 2, 16
NW = NC * NS
CHUNK = 128
LANES = 16
ROWS_PER_W = B // (NW * CHUNK)


def _sc_gather(params_flat, i2d, j2d):
    mesh = plsc.VectorSubcoreMesh(
        core_axis_name="c", subcore_axis_name="s",
        num_cores=NC, num_subcores=NS,
    )

    @functools.partial(
        pl.kernel,
        out_type=jax.ShapeDtypeStruct((B // CHUNK, CHUNK), jnp.float32),
        mesh=mesh,
        scratch_types=[
            pltpu.VMEM((ROWS_PER_W, CHUNK), jnp.int32),
            pltpu.VMEM((ROWS_PER_W, CHUNK), jnp.int32),
            pltpu.VMEM((ROWS_PER_W, CHUNK), jnp.float32),
            pltpu.SemaphoreType.DMA,
        ],
    )
    def k(params_hbm, i_hbm, j_hbm, out_hbm, i_v, idx_v, vals_v, sem):
        wid = lax.axis_index("s") * NC + lax.axis_index("c")
        base = wid * ROWS_PER_W
        pltpu.sync_copy(i_hbm.at[pl.ds(base, ROWS_PER_W)], i_v)
        pltpu.sync_copy(j_hbm.at[pl.ds(base, ROWS_PER_W)], idx_v)
        for c in range(ROWS_PER_W):
            for s in range(CHUNK // LANES):
                sl = (c, pl.ds(s * LANES, LANES))
                idx_v[sl] = i_v[sl] * N + idx_v[sl]
        cps = [
            pltpu.async_copy(params_hbm.at[idx_v.at[c]], vals_v.at[c], sem)
            for c in range(ROWS_PER_W)
        ]
        for cp in cps:
            cp.wait()
        pltpu.sync_copy(vals_v, out_hbm.at[pl.ds(base, ROWS_PER_W)])

    return k(params_flat, i2d, j2d)


def kernel(params, i, j):
    params_flat = params.reshape(N * N)
    i2d = i.astype(jnp.int32).reshape(B // CHUNK, CHUNK)
    j2d = j.astype(jnp.int32).reshape(B // CHUNK, CHUNK)
    out = _sc_gather(params_flat, i2d, j2d)
    return out.reshape(B)


# speedup vs baseline: 9.4865x; 9.4865x over previous
import functools

import jax
import jax.numpy as jnp
from jax import lax
from jax.experimental import pallas as pl
from jax.experimental.pallas import tpu as pltpu
from jax.experimental.pallas import tpu_sc as plsc

N = 8192
B = 16384
NC, NS = 2, 16
NW = NC * NS
CHUNK = 128
LANES = 16
ROWS_PER_W = B // (NW * CHUNK)


def _sc_gather(params_flat, i2d, j2d):
    mesh = plsc.VectorSubcoreMesh(
        core_axis_name="c", subcore_axis_name="s",
        num_cores=NC, num_subcores=NS,
    )

    @functools.partial(
        pl.kernel,
        out_type=jax.ShapeDtypeStruct((B // CHUNK, CHUNK), jnp.float32),
        mesh=mesh,
        scratch_types=[
            pltpu.VMEM((ROWS_PER_W, CHUNK), jnp.int32),
            pltpu.VMEM((ROWS_PER_W, CHUNK), jnp.int32),
            pltpu.VMEM((ROWS_PER_W, CHUNK), jnp.float32),
            pltpu.SemaphoreType.DMA,
        ],
    )
    def k(params_hbm, i_hbm, j_hbm, out_hbm, i_v, idx_v, vals_v, sem):
        wid = lax.axis_index("s") * NC + lax.axis_index("c")
        base = wid * ROWS_PER_W
        pltpu.sync_copy(i_hbm.at[pl.ds(base, ROWS_PER_W)], i_v)
        pltpu.sync_copy(j_hbm.at[pl.ds(base, ROWS_PER_W)], idx_v)
        for c in range(ROWS_PER_W):
            for s in range(CHUNK // LANES):
                sl = (c, pl.ds(s * LANES, LANES))
                iv = i_v[sl]
                jv = idx_v[sl]
                idx_v[sl] = (
                    ((iv >> 3) << 16)
                    + ((jv >> 7) << 10)
                    + ((iv & 7) << 7)
                    + (jv & 127)
                )
        cps = [
            pltpu.async_copy(params_hbm.at[idx_v.at[c]], vals_v.at[c], sem)
            for c in range(ROWS_PER_W)
        ]
        for cp in cps:
            cp.wait()
        pltpu.sync_copy(vals_v, out_hbm.at[pl.ds(base, ROWS_PER_W)])

    return k(params_flat, i2d, j2d)


def kernel(params, i, j):
    params_flat = (
        params.reshape(N // 8, 8, N // 128, 128)
        .transpose(0, 2, 1, 3)
        .reshape(N * N)
    )
    i2d = i.astype(jnp.int32).reshape(B // CHUNK, CHUNK)
    j2d = j.astype(jnp.int32).reshape(B // CHUNK, CHUNK)
    out = _sc_gather(params_flat, i2d, j2d)
    return out.reshape(B)


# speedup vs baseline: 9.6478x; 1.0170x over previous
import functools

import jax
import jax.numpy as jnp
from jax import lax
from jax.experimental import pallas as pl
from jax.experimental.pallas import tpu as pltpu
from jax.experimental.pallas import tpu_sc as plsc

N = 8192
B = 16384
NC, NS = 1, 16
NW = NC * NS
CHUNK = 128
LANES = 16
ROWS_PER_W = B // (NW * CHUNK)


def _sc_gather(params_flat, i2d, j2d):
    mesh = plsc.VectorSubcoreMesh(
        core_axis_name="c", subcore_axis_name="s",
        num_cores=NC, num_subcores=NS,
    )

    @functools.partial(
        pl.kernel,
        out_type=jax.ShapeDtypeStruct((B // CHUNK, CHUNK), jnp.float32),
        mesh=mesh,
        scratch_types=[
            pltpu.VMEM((ROWS_PER_W, CHUNK), jnp.int32),
            pltpu.VMEM((ROWS_PER_W, CHUNK), jnp.int32),
            pltpu.VMEM((ROWS_PER_W, CHUNK), jnp.float32),
            pltpu.SemaphoreType.DMA,
        ],
    )
    def k(params_hbm, i_hbm, j_hbm, out_hbm, i_v, idx_v, vals_v, sem):
        wid = lax.axis_index("s") * NC + lax.axis_index("c")
        base = wid * ROWS_PER_W
        pltpu.sync_copy(i_hbm.at[pl.ds(base, ROWS_PER_W)], i_v)
        pltpu.sync_copy(j_hbm.at[pl.ds(base, ROWS_PER_W)], idx_v)
        for c in range(ROWS_PER_W):
            for s in range(CHUNK // LANES):
                sl = (c, pl.ds(s * LANES, LANES))
                iv = i_v[sl]
                jv = idx_v[sl]
                idx_v[sl] = (
                    ((iv >> 3) << 16)
                    + ((jv >> 7) << 10)
                    + ((iv & 7) << 7)
                    + (jv & 127)
                )
        cps = [
            pltpu.async_copy(params_hbm.at[idx_v.at[c]], vals_v.at[c], sem)
            for c in range(ROWS_PER_W)
        ]
        for cp in cps:
            cp.wait()
        pltpu.sync_copy(vals_v, out_hbm.at[pl.ds(base, ROWS_PER_W)])

    return k(params_flat, i2d, j2d)


def kernel(params, i, j):
    params_flat = (
        params.reshape(N // 8, 8, N // 128, 128)
        .transpose(0, 2, 1, 3)
        .reshape(N * N)
    )
    i2d = i.astype(jnp.int32).reshape(B // CHUNK, CHUNK)
    j2d = j.astype(jnp.int32).reshape(B // CHUNK, CHUNK)
    out = _sc_gather(params_flat, i2d, j2d)
    return out.reshape(B)
